# Initial kernel scaffold; baseline (speedup 1.0000x reference)
import jax
import jax.numpy as jnp
from jax.experimental import pallas as pl


def kernel(x, edge_index, Wl1, bl1, Wr1, Wl2, bl2, Wr2, Wout, bout):
    raise NotImplementedError("write your pallas kernel here")


# speedup vs baseline: 2.5331x; 2.5331x over previous
import functools

import jax
import jax.numpy as jnp
from jax import lax
from jax.experimental import pallas as pl
from jax.experimental.pallas import tpu as pltpu
from jax.experimental.pallas import tpu_sc as plsc

N = 10000
D = 128
E = 320000

NC = 2
NS = 16
CHUNK = 80
EPS = E // NS
ROWS_S = EPS // CHUNK
NHALF = 5120
ACC_ROWS = 5248
GARBAGE = NHALF
ZROWS_S = ACC_ROWS // NS
OUT_S = NHALF // NS
CW = 128

_mesh = plsc.VectorSubcoreMesh(core_axis_name="c", subcore_axis_name="s")


def _zero_chunks(total, step):
    out = []
    o = 0
    while o < total:
        ln = min(step, total - o)
        out.append((o, ln))
        o += ln
    return out


def _remap_dst(dst_v, c):
    base = c * NHALF
    for o in range(0, CHUNK, 16):
        v = dst_v[pl.ds(o, 16)] - base
        ok = (v >= 0) & (v < NHALF)
        dst_v[pl.ds(o, 16)] = jnp.where(ok, v, GARBAGE)


def _segment_sums(feat, src_flat, dst_flat):
    @functools.partial(
        pl.kernel,
        out_type=jax.ShapeDtypeStruct((NC, NHALF, D), jnp.float32),
        mesh=_mesh,
        scratch_types=[
            pltpu.VMEM((CHUNK,), jnp.int32),
            pltpu.VMEM((CHUNK,), jnp.int32),
            pltpu.VMEM((CHUNK, D), jnp.float32),
            pltpu.VMEM_SHARED((ACC_ROWS, D), jnp.float32),
            pltpu.SemaphoreType.DMA,
        ],
    )
    def k(feat_hbm, src_hbm, dst_hbm, sums_hbm,
          src_v, dst_v, rows_v, acc_sh, sem):
        c = lax.axis_index("c")
        s = lax.axis_index("s")
        base_e = s * EPS

        zvec = jnp.zeros((16,), jnp.float32)

        @pl.loop(0, CHUNK)
        def _(r):
            @pl.loop(0, D, step=16)
            def _(cc):
                rows_v[r, pl.ds(cc, 16)] = zvec

        zbase = s * ZROWS_S
        for off, ln in _zero_chunks(ZROWS_S, CHUNK):
            pltpu.sync_copy(rows_v.at[pl.ds(0, ln)],
                            acc_sh.at[pl.ds(zbase + off, ln)])
        plsc.subcore_barrier()

        @pl.loop(0, ROWS_S)
        def _(i):
            e0 = base_e + i * CHUNK
            pltpu.sync_copy(src_hbm.at[pl.ds(e0, CHUNK)], src_v)
            pltpu.sync_copy(dst_hbm.at[pl.ds(e0, CHUNK)], dst_v)
            _remap_dst(dst_v, c)
            pltpu.async_copy(feat_hbm.at[src_v], rows_v, sem).wait()
            pltpu.sync_copy(rows_v, acc_sh.at[dst_v], add=True)

        plsc.subcore_barrier()
        obase = s * OUT_S
        pltpu.sync_copy(acc_sh.at[pl.ds(obase, OUT_S)],
                        sums_hbm.at[c].at[pl.ds(obase, OUT_S)])

    return k(feat, src_flat, dst_flat).reshape(NC * NHALF, D)


def _degree_counts(dst_flat):
    @functools.partial(
        pl.kernel,
        out_type=jax.ShapeDtypeStruct((NC, NHALF, CW), jnp.float32),
        mesh=_mesh,
        scratch_types=[
            pltpu.VMEM((CHUNK,), jnp.int32),
            pltpu.VMEM((CHUNK, CW), jnp.float32),
            pltpu.VMEM_SHARED((ACC_ROWS, CW), jnp.float32),
        ],
    )
    def k(dst_hbm, cnts_hbm, dst_v, ones_v, cnt_sh):
        c = lax.axis_index("c")
        s = lax.axis_index("s")
        base_e = s * EPS

        zvec = jnp.zeros((16,), jnp.float32)

        @pl.loop(0, CHUNK)
        def _(r):
            @pl.loop(0, CW, step=16)
            def _(cc):
                ones_v[r, pl.ds(cc, 16)] = zvec

        zbase = s * ZROWS_S
        for off, ln in _zero_chunks(ZROWS_S, CHUNK):
            pltpu.sync_copy(ones_v.at[pl.ds(0, ln)],
                            cnt_sh.at[pl.ds(zbase + off, ln)])
        plsc.subcore_barrier()

        ovec = jnp.full((16,), 1.0, jnp.float32)

        @pl.loop(0, CHUNK)
        def _(r):
            @pl.loop(0, CW, step=16)
            def _(cc):
                ones_v[r, pl.ds(cc, 16)] = ovec

        @pl.loop(0, ROWS_S)
        def _(i):
            e0 = base_e + i * CHUNK
            pltpu.sync_copy(dst_hbm.at[pl.ds(e0, CHUNK)], dst_v)
            _remap_dst(dst_v, c)
            pltpu.sync_copy(ones_v, cnt_sh.at[dst_v], add=True)

        plsc.subcore_barrier()
        obase = s * OUT_S
        pltpu.sync_copy(cnt_sh.at[pl.ds(obase, OUT_S)],
                        cnts_hbm.at[c].at[pl.ds(obase, OUT_S)])

    return k(dst_flat).reshape(NC * NHALF, CW)


BN = 1000


def _tc_layer(sums, cnts, feat, Wl, bl, Wr, head=None):
    with_head = head is not None

    def body(*refs):
        if with_head:
            (sums_ref, cnt_ref, x_ref, wl_ref, bl_ref, wr_ref,
             wo_ref, bo_ref, o_ref) = refs
        else:
            (sums_ref, cnt_ref, x_ref, wl_ref, bl_ref, wr_ref,
             o_ref) = refs
        cnt = cnt_ref[:, 0:1]
        mean = sums_ref[...] / jnp.maximum(cnt, 1.0)
        h = (jnp.dot(mean, wl_ref[...], precision=lax.Precision.HIGHEST)
             + bl_ref[...]
             + jnp.dot(x_ref[...], wr_ref[...], precision=lax.Precision.HIGHEST))
        h = jnp.where(h >= 0, h, 0.01 * h)
        if with_head:
            o_ref[...] = (jnp.dot(h, wo_ref[...], precision=lax.Precision.HIGHEST)
                          + bo_ref[...])
        else:
            o_ref[...] = h

    in_specs = [
        pl.BlockSpec((BN, D), lambda i: (i, 0)),
        pl.BlockSpec((BN, CW), lambda i: (i, 0)),
        pl.BlockSpec((BN, D), lambda i: (i, 0)),
        pl.BlockSpec((D, D), lambda i: (0, 0)),
        pl.BlockSpec((1, D), lambda i: (0, 0)),
        pl.BlockSpec((D, D), lambda i: (0, 0)),
    ]
    args = [sums, cnts, feat, Wl, bl.reshape(1, D), Wr]
    if with_head:
        Wout, bout = head
        in_specs += [
            pl.BlockSpec((D, 1), lambda i: (0, 0)),
            pl.BlockSpec((1, 1), lambda i: (0, 0)),
        ]
        args += [Wout, bout.reshape(1, 1)]
        out_spec = pl.BlockSpec((BN, 1), lambda i: (i, 0))
        out_shape = jax.ShapeDtypeStruct((N, 1), jnp.float32)
    else:
        out_spec = pl.BlockSpec((BN, D), lambda i: (i, 0))
        out_shape = jax.ShapeDtypeStruct((N, D), jnp.float32)

    return pl.pallas_call(
        body,
        grid=(N // BN,),
        in_specs=in_specs,
        out_specs=out_spec,
        out_shape=out_shape,
    )(*args)


def kernel(x, edge_index, Wl1, bl1, Wr1, Wl2, bl2, Wr2, Wout, bout):
    src_flat = edge_index[0]
    dst_flat = edge_index[1]

    cnts = _degree_counts(dst_flat)

    sums1 = _segment_sums(x, src_flat, dst_flat)
    h1 = _tc_layer(sums1, cnts, x, Wl1, bl1, Wr1)

    sums2 = _segment_sums(h1, src_flat, dst_flat)
    out = _tc_layer(sums2, cnts, h1, Wl2, bl2, Wr2, head=(Wout, bout))
    return out.reshape(N)


# speedup vs baseline: 3.7503x; 1.4805x over previous
import functools

import jax
import jax.numpy as jnp
from jax import lax
from jax.experimental import pallas as pl
from jax.experimental.pallas import tpu as pltpu
from jax.experimental.pallas import tpu_sc as plsc

N = 10000
D = 128
E = 320000

NC = 2
NS = 16
CHUNK = 80
EPS = E // NS
ROWS_S = EPS // CHUNK
NHALF = 5120
ACC_ROWS = 5248
GARBAGE = NHALF
ZROWS_S = ACC_ROWS // NS
OUT_S = NHALF // NS
CW = 128

_mesh = plsc.VectorSubcoreMesh(core_axis_name="c", subcore_axis_name="s")


def _zero_chunks(total, step):
    out = []
    o = 0
    while o < total:
        ln = min(step, total - o)
        out.append((o, ln))
        o += ln
    return out


def _remap_dst(dst_v, c):
    base = c * NHALF
    for o in range(0, CHUNK, 16):
        v = dst_v[pl.ds(o, 16)] - base
        ok = (v >= 0) & (v < NHALF)
        dst_v[pl.ds(o, 16)] = jnp.where(ok, v, GARBAGE)


def _segment_sums(feat, src_flat, dst_flat):
    @functools.partial(
        pl.kernel,
        out_type=jax.ShapeDtypeStruct((NC, NHALF, D), jnp.float32),
        mesh=_mesh,
        scratch_types=[
            pltpu.VMEM((CHUNK,), jnp.int32),
            pltpu.VMEM((CHUNK,), jnp.int32),
            pltpu.VMEM((CHUNK, D), jnp.float32),
            pltpu.VMEM((CHUNK,), jnp.int32),
            pltpu.VMEM((CHUNK,), jnp.int32),
            pltpu.VMEM((CHUNK, D), jnp.float32),
            pltpu.VMEM_SHARED((ACC_ROWS, D), jnp.float32),
            pltpu.SemaphoreType.DMA,
            pltpu.SemaphoreType.DMA,
        ],
    )
    def k(feat_hbm, src_hbm, dst_hbm, sums_hbm,
          src0_v, dst0_v, rows0_v, src1_v, dst1_v, rows1_v,
          acc_sh, sem0, sem1):
        c = lax.axis_index("c")
        s = lax.axis_index("s")
        base_e = s * EPS

        zvec = jnp.zeros((16,), jnp.float32)

        @pl.loop(0, CHUNK)
        def _(r):
            @pl.loop(0, D, step=16)
            def _(cc):
                rows0_v[r, pl.ds(cc, 16)] = zvec

        zbase = s * ZROWS_S
        for off, ln in _zero_chunks(ZROWS_S, CHUNK):
            pltpu.sync_copy(rows0_v.at[pl.ds(0, ln)],
                            acc_sh.at[pl.ds(zbase + off, ln)])
        plsc.subcore_barrier()

        def load_idx(j, src_v, dst_v):
            e0 = base_e + j * CHUNK
            pltpu.sync_copy(src_hbm.at[pl.ds(e0, CHUNK)], src_v)
            pltpu.sync_copy(dst_hbm.at[pl.ds(e0, CHUNK)], dst_v)
            _remap_dst(dst_v, c)

        def fire(src_v, rows_v, sem):
            pltpu.async_copy(feat_hbm.at[src_v], rows_v, sem)

        def drain_scatter(src_v, rows_v, dst_v, sem):
            pltpu.make_async_copy(feat_hbm.at[src_v], rows_v, sem).wait()
            pltpu.sync_copy(rows_v, acc_sh.at[dst_v], add=True)

        load_idx(0, src0_v, dst0_v)
        fire(src0_v, rows0_v, sem0)

        @pl.loop(0, ROWS_S - 2, step=2)
        def _(i):
            load_idx(i + 1, src1_v, dst1_v)
            fire(src1_v, rows1_v, sem1)
            drain_scatter(src0_v, rows0_v, dst0_v, sem0)
            load_idx(i + 2, src0_v, dst0_v)
            fire(src0_v, rows0_v, sem0)
            drain_scatter(src1_v, rows1_v, dst1_v, sem1)

        load_idx(ROWS_S - 1, src1_v, dst1_v)
        fire(src1_v, rows1_v, sem1)
        drain_scatter(src0_v, rows0_v, dst0_v, sem0)
        drain_scatter(src1_v, rows1_v, dst1_v, sem1)

        plsc.subcore_barrier()
        obase = s * OUT_S
        pltpu.sync_copy(acc_sh.at[pl.ds(obase, OUT_S)],
                        sums_hbm.at[c].at[pl.ds(obase, OUT_S)])

    return k(feat, src_flat, dst_flat).reshape(NC * NHALF, D)


def _degree_counts(dst_flat):
    @functools.partial(
        pl.kernel,
        out_type=jax.ShapeDtypeStruct((NC, NHALF, CW), jnp.float32),
        mesh=_mesh,
        scratch_types=[
            pltpu.VMEM((CHUNK,), jnp.int32),
            pltpu.VMEM((CHUNK, CW), jnp.float32),
            pltpu.VMEM_SHARED((ACC_ROWS, CW), jnp.float32),
        ],
    )
    def k(dst_hbm, cnts_hbm, dst_v, ones_v, cnt_sh):
        c = lax.axis_index("c")
        s = lax.axis_index("s")
        base_e = s * EPS

        zvec = jnp.zeros((16,), jnp.float32)

        @pl.loop(0, CHUNK)
        def _(r):
            @pl.loop(0, CW, step=16)
            def _(cc):
                ones_v[r, pl.ds(cc, 16)] = zvec

        zbase = s * ZROWS_S
        for off, ln in _zero_chunks(ZROWS_S, CHUNK):
            pltpu.sync_copy(ones_v.at[pl.ds(0, ln)],
                            cnt_sh.at[pl.ds(zbase + off, ln)])
        plsc.subcore_barrier()

        ovec = jnp.full((16,), 1.0, jnp.float32)

        @pl.loop(0, CHUNK)
        def _(r):
            @pl.loop(0, CW, step=16)
            def _(cc):
                ones_v[r, pl.ds(cc, 16)] = ovec

        @pl.loop(0, ROWS_S)
        def _(i):
            e0 = base_e + i * CHUNK
            pltpu.sync_copy(dst_hbm.at[pl.ds(e0, CHUNK)], dst_v)
            _remap_dst(dst_v, c)
            pltpu.sync_copy(ones_v, cnt_sh.at[dst_v], add=True)

        plsc.subcore_barrier()
        obase = s * OUT_S
        pltpu.sync_copy(cnt_sh.at[pl.ds(obase, OUT_S)],
                        cnts_hbm.at[c].at[pl.ds(obase, OUT_S)])

    return k(dst_flat).reshape(NC * NHALF, CW)


BN = 1000


def _tc_layer(sums, cnts, feat, Wl, bl, Wr, head=None):
    with_head = head is not None

    def body(*refs):
        if with_head:
            (sums_ref, cnt_ref, x_ref, wl_ref, bl_ref, wr_ref,
             wo_ref, bo_ref, o_ref) = refs
        else:
            (sums_ref, cnt_ref, x_ref, wl_ref, bl_ref, wr_ref,
             o_ref) = refs
        cnt = cnt_ref[:, 0:1]
        mean = sums_ref[...] / jnp.maximum(cnt, 1.0)
        h = (jnp.dot(mean, wl_ref[...], precision=lax.Precision.HIGHEST)
             + bl_ref[...]
             + jnp.dot(x_ref[...], wr_ref[...], precision=lax.Precision.HIGHEST))
        h = jnp.where(h >= 0, h, 0.01 * h)
        if with_head:
            o_ref[...] = (jnp.dot(h, wo_ref[...], precision=lax.Precision.HIGHEST)
                          + bo_ref[...])
        else:
            o_ref[...] = h

    in_specs = [
        pl.BlockSpec((BN, D), lambda i: (i, 0)),
        pl.BlockSpec((BN, CW), lambda i: (i, 0)),
        pl.BlockSpec((BN, D), lambda i: (i, 0)),
        pl.BlockSpec((D, D), lambda i: (0, 0)),
        pl.BlockSpec((1, D), lambda i: (0, 0)),
        pl.BlockSpec((D, D), lambda i: (0, 0)),
    ]
    args = [sums, cnts, feat, Wl, bl.reshape(1, D), Wr]
    if with_head:
        Wout, bout = head
        in_specs += [
            pl.BlockSpec((D, 1), lambda i: (0, 0)),
            pl.BlockSpec((1, 1), lambda i: (0, 0)),
        ]
        args += [Wout, bout.reshape(1, 1)]
        out_spec = pl.BlockSpec((BN, 1), lambda i: (i, 0))
        out_shape = jax.ShapeDtypeStruct((N, 1), jnp.float32)
    else:
        out_spec = pl.BlockSpec((BN, D), lambda i: (i, 0))
        out_shape = jax.ShapeDtypeStruct((N, D), jnp.float32)

    return pl.pallas_call(
        body,
        grid=(N // BN,),
        in_specs=in_specs,
        out_specs=out_spec,
        out_shape=out_shape,
    )(*args)


def kernel(x, edge_index, Wl1, bl1, Wr1, Wl2, bl2, Wr2, Wout, bout):
    src_flat = edge_index[0]
    dst_flat = edge_index[1]

    cnts = _degree_counts(dst_flat)

    sums1 = _segment_sums(x, src_flat, dst_flat)
    h1 = _tc_layer(sums1, cnts, x, Wl1, bl1, Wr1)

    sums2 = _segment_sums(h1, src_flat, dst_flat)
    out = _tc_layer(sums2, cnts, h1, Wl2, bl2, Wr2, head=(Wout, bout))
    return out.reshape(N)


# speedup vs baseline: 4.1040x; 1.0943x over previous
import functools

import jax
import jax.numpy as jnp
from jax import lax
from jax.experimental import pallas as pl
from jax.experimental.pallas import tpu as pltpu
from jax.experimental.pallas import tpu_sc as plsc

N = 10000
D = 128
E = 320000

NC = 2
NS = 16
CHUNK = 80
EPS = E // NS
ROWS_S = EPS // CHUNK
NHALF = 5120
ACC_ROWS = 5248
GARBAGE = NHALF
ZROWS_S = ACC_ROWS // NS
OUT_S = NHALF // NS
CW = 128

_mesh = plsc.VectorSubcoreMesh(core_axis_name="c", subcore_axis_name="s")


def _zero_chunks(total, step):
    out = []
    o = 0
    while o < total:
        ln = min(step, total - o)
        out.append((o, ln))
        o += ln
    return out


def _remap_dst(dst_v, c):
    base = c * NHALF
    for o in range(0, CHUNK, 16):
        v = dst_v[pl.ds(o, 16)] - base
        ok = (v >= 0) & (v < NHALF)
        dst_v[pl.ds(o, 16)] = jnp.where(ok, v, GARBAGE)


def _segment_sums(feat, src_flat, dst_flat, with_counts=False):
    out_type = [jax.ShapeDtypeStruct((NC, NHALF, D), jnp.float32)]
    scratch = [
        pltpu.VMEM((CHUNK,), jnp.int32),
        pltpu.VMEM((CHUNK,), jnp.int32),
        pltpu.VMEM((CHUNK, D), jnp.float32),
        pltpu.VMEM((CHUNK,), jnp.int32),
        pltpu.VMEM((CHUNK,), jnp.int32),
        pltpu.VMEM((CHUNK, D), jnp.float32),
        pltpu.VMEM_SHARED((ACC_ROWS, D), jnp.float32),
        pltpu.SemaphoreType.DMA,
        pltpu.SemaphoreType.DMA,
    ]
    if with_counts:
        out_type.append(jax.ShapeDtypeStruct((NC, NHALF, D), jnp.float32))
        scratch += [
            pltpu.VMEM((CHUNK, D), jnp.float32),
            pltpu.VMEM_SHARED((ACC_ROWS, D), jnp.float32),
        ]

    @functools.partial(
        pl.kernel,
        out_type=out_type,
        mesh=_mesh,
        scratch_types=scratch,
    )
    def k(feat_hbm, src_hbm, dst_hbm, *out_and_scratch):
        if with_counts:
            (sums_hbm, cnts_hbm,
             src0_v, dst0_v, rows0_v, src1_v, dst1_v, rows1_v,
             acc_sh, sem0, sem1, ones_v, cnt_sh) = out_and_scratch
        else:
            (sums_hbm,
             src0_v, dst0_v, rows0_v, src1_v, dst1_v, rows1_v,
             acc_sh, sem0, sem1) = out_and_scratch
        c = lax.axis_index("c")
        s = lax.axis_index("s")
        base_e = s * EPS

        zvec = jnp.zeros((16,), jnp.float32)

        @pl.loop(0, CHUNK)
        def _(r):
            @pl.loop(0, D, step=16)
            def _(cc):
                rows0_v[r, pl.ds(cc, 16)] = zvec

        zbase = s * ZROWS_S
        for off, ln in _zero_chunks(ZROWS_S, CHUNK):
            pltpu.sync_copy(rows0_v.at[pl.ds(0, ln)],
                            acc_sh.at[pl.ds(zbase + off, ln)])
            if with_counts:
                pltpu.sync_copy(rows0_v.at[pl.ds(0, ln)],
                                cnt_sh.at[pl.ds(zbase + off, ln)])
        if with_counts:
            ovec = jnp.full((16,), 1.0, jnp.float32)

            @pl.loop(0, CHUNK)
            def _(r):
                @pl.loop(0, D, step=16)
                def _(cc):
                    ones_v[r, pl.ds(cc, 16)] = ovec
        plsc.subcore_barrier()

        def load_idx(j, src_v, dst_v):
            e0 = base_e + j * CHUNK
            pltpu.sync_copy(src_hbm.at[pl.ds(e0, CHUNK)], src_v)
            pltpu.sync_copy(dst_hbm.at[pl.ds(e0, CHUNK)], dst_v)
            _remap_dst(dst_v, c)

        def fire(src_v, rows_v, sem):
            pltpu.async_copy(feat_hbm.at[src_v], rows_v, sem)

        def drain_scatter(src_v, rows_v, dst_v, sem):
            pltpu.make_async_copy(feat_hbm.at[src_v], rows_v, sem).wait()
            pltpu.sync_copy(rows_v, acc_sh.at[dst_v], add=True)
            if with_counts:
                pltpu.sync_copy(ones_v, cnt_sh.at[dst_v], add=True)

        load_idx(0, src0_v, dst0_v)
        fire(src0_v, rows0_v, sem0)

        @pl.loop(0, ROWS_S - 2, step=2)
        def _(i):
            load_idx(i + 1, src1_v, dst1_v)
            fire(src1_v, rows1_v, sem1)
            drain_scatter(src0_v, rows0_v, dst0_v, sem0)
            load_idx(i + 2, src0_v, dst0_v)
            fire(src0_v, rows0_v, sem0)
            drain_scatter(src1_v, rows1_v, dst1_v, sem1)

        load_idx(ROWS_S - 1, src1_v, dst1_v)
        fire(src1_v, rows1_v, sem1)
        drain_scatter(src0_v, rows0_v, dst0_v, sem0)
        drain_scatter(src1_v, rows1_v, dst1_v, sem1)

        plsc.subcore_barrier()
        obase = s * OUT_S
        pltpu.sync_copy(acc_sh.at[pl.ds(obase, OUT_S)],
                        sums_hbm.at[c].at[pl.ds(obase, OUT_S)])
        if with_counts:
            pltpu.sync_copy(cnt_sh.at[pl.ds(obase, OUT_S)],
                            cnts_hbm.at[c].at[pl.ds(obase, OUT_S)])

    res = k(feat, src_flat, dst_flat)
    if with_counts:
        sums, cnts = res
        return (sums.reshape(NC * NHALF, D), cnts.reshape(NC * NHALF, D))
    return res[0].reshape(NC * NHALF, D)


BN = 1000


def _tc_layer(sums, cnts, feat, Wl, bl, Wr, head=None):
    with_head = head is not None

    def body(*refs):
        if with_head:
            (sums_ref, cnt_ref, x_ref, wl_ref, bl_ref, wr_ref,
             wo_ref, bo_ref, o_ref) = refs
        else:
            (sums_ref, cnt_ref, x_ref, wl_ref, bl_ref, wr_ref,
             o_ref) = refs
        cnt = cnt_ref[:, 0:1]
        mean = sums_ref[...] / jnp.maximum(cnt, 1.0)
        h = (jnp.dot(mean, wl_ref[...], precision=lax.Precision.HIGHEST)
             + bl_ref[...]
             + jnp.dot(x_ref[...], wr_ref[...], precision=lax.Precision.HIGHEST))
        h = jnp.where(h >= 0, h, 0.01 * h)
        if with_head:
            o_ref[...] = (jnp.dot(h, wo_ref[...], precision=lax.Precision.HIGHEST)
                          + bo_ref[...])
        else:
            o_ref[...] = h

    in_specs = [
        pl.BlockSpec((BN, D), lambda i: (i, 0)),
        pl.BlockSpec((BN, CW), lambda i: (i, 0)),
        pl.BlockSpec((BN, D), lambda i: (i, 0)),
        pl.BlockSpec((D, D), lambda i: (0, 0)),
        pl.BlockSpec((1, D), lambda i: (0, 0)),
        pl.BlockSpec((D, D), lambda i: (0, 0)),
    ]
    args = [sums, cnts, feat, Wl, bl.reshape(1, D), Wr]
    if with_head:
        Wout, bout = head
        in_specs += [
            pl.BlockSpec((D, 1), lambda i: (0, 0)),
            pl.BlockSpec((1, 1), lambda i: (0, 0)),
        ]
        args += [Wout, bout.reshape(1, 1)]
        out_spec = pl.BlockSpec((BN, 1), lambda i: (i, 0))
        out_shape = jax.ShapeDtypeStruct((N, 1), jnp.float32)
    else:
        out_spec = pl.BlockSpec((BN, D), lambda i: (i, 0))
        out_shape = jax.ShapeDtypeStruct((N, D), jnp.float32)

    return pl.pallas_call(
        body,
        grid=(N // BN,),
        in_specs=in_specs,
        out_specs=out_spec,
        out_shape=out_shape,
    )(*args)


def kernel(x, edge_index, Wl1, bl1, Wr1, Wl2, bl2, Wr2, Wout, bout):
    src_flat = edge_index[0]
    dst_flat = edge_index[1]

    sums1, cnts = _segment_sums(x, src_flat, dst_flat, with_counts=True)
    h1 = _tc_layer(sums1, cnts, x, Wl1, bl1, Wr1)

    sums2 = _segment_sums(h1, src_flat, dst_flat)
    out = _tc_layer(sums2, cnts, h1, Wl2, bl2, Wr2, head=(Wout, bout))
    return out.reshape(N)
